# baseline (speedup 1.0000x reference)
import functools

import jax
import jax.numpy as jnp
from jax import lax
from jax.experimental import pallas as pl
from jax.experimental.pallas import tpu as pltpu
from jax.experimental.pallas import tpu_sc as plsc

NUM_EMB = 100000
EMB_DIM = 128
PROJ_DIM = 256
DENSE_DIM = 64
CH = 32
HW = 256
OUT_DIM = 64
B = 4096
EPS = 1e-05

_NUM_SC = 2
_NUM_SUBCORES = 16
_NW = _NUM_SC * _NUM_SUBCORES


def _sc_gather(table, idx):
    b_per_w = B // _NW
    mesh = plsc.VectorSubcoreMesh(core_axis_name="c", subcore_axis_name="s")

    @functools.partial(
        pl.kernel,
        mesh=mesh,
        out_type=jax.ShapeDtypeStruct((B, EMB_DIM), jnp.float32),
        scratch_types=[
            pltpu.VMEM((b_per_w,), jnp.int32),
            pltpu.VMEM((b_per_w, EMB_DIM), jnp.float32),
            pltpu.SemaphoreType.DMA,
        ],
    )
    def gather_kernel(table_hbm, idx_hbm, out_hbm, idx_v, rows_v, sem):
        wid = lax.axis_index("s") * _NUM_SC + lax.axis_index("c")
        base = wid * b_per_w
        pltpu.sync_copy(idx_hbm.at[pl.ds(base, b_per_w)], idx_v)
        pltpu.async_copy(table_hbm.at[idx_v], rows_v, sem).wait()
        pltpu.sync_copy(rows_v, out_hbm.at[pl.ds(base, b_per_w)])

    return gather_kernel(table, idx)


def _fused_body(g_ref, d_ref, wproj_ref, bproj_ref, wsp_ref, bsp_ref,
                gam_ref, bet_ref, whT_ref, bh_ref, out_ref):
    c = pl.program_id(0)
    w_blk = wsp_ref[...]
    wp = w_blk[:, :PROJ_DIM]
    wd = w_blk[:, PROJ_DIM:].astype(jnp.bfloat16)
    wp_b = wp.astype(jnp.bfloat16)
    m = lax.dot_general(wp_b, wproj_ref[...].astype(jnp.bfloat16),
                        (((1,), (0,)), ((), ())),
                        preferred_element_type=jnp.float32
                        ).astype(jnp.bfloat16)
    s = lax.dot_general(g_ref[...].astype(jnp.bfloat16), m,
                        (((1,), (1,)), ((), ())),
                        preferred_element_type=jnp.float32)
    s = s + lax.dot_general(d_ref[...].astype(jnp.bfloat16), wd,
                            (((1,), (1,)), ((), ())),
                            preferred_element_type=jnp.float32)
    brow = lax.dot_general(bproj_ref[...], wp, (((1,), (1,)), ((), ())),
                           preferred_element_type=jnp.float32)
    s = s + (brow + bsp_ref[...])
    mu = jnp.mean(s, axis=1, keepdims=True)
    ms2 = jnp.mean(s * s, axis=1, keepdims=True)
    var = ms2 - mu * mu
    a = lax.rsqrt(var + EPS) * gam_ref[0, c]
    bb = bet_ref[0, c] - a * mu
    y = jnp.maximum(a * s + bb, 0.0)
    pooled = jnp.mean(y, axis=1, keepdims=True)
    contrib = pooled * whT_ref[pl.ds(c, 1), :]

    @pl.when(c == 0)
    def _():
        out_ref[...] = contrib + bh_ref[...]

    @pl.when(c > 0)
    def _():
        out_ref[...] += contrib


def _fused_dense(g, dense, W_proj, b_proj2, W_sp, b_sp2, gammaB, betaB,
                 W_headT, b_head2):
    return pl.pallas_call(
        _fused_body,
        grid=(CH,),
        in_specs=[
            pl.BlockSpec((B, EMB_DIM), lambda c: (0, 0)),
            pl.BlockSpec((B, DENSE_DIM), lambda c: (0, 0)),
            pl.BlockSpec((PROJ_DIM, EMB_DIM), lambda c: (0, 0)),
            pl.BlockSpec((1, PROJ_DIM), lambda c: (0, 0)),
            pl.BlockSpec((HW, PROJ_DIM + DENSE_DIM), lambda c: (c, 0)),
            pl.BlockSpec((1, HW), lambda c: (0, c)),
            pl.BlockSpec(memory_space=pltpu.SMEM),
            pl.BlockSpec(memory_space=pltpu.SMEM),
            pl.BlockSpec((CH, OUT_DIM), lambda c: (0, 0)),
            pl.BlockSpec((1, OUT_DIM), lambda c: (0, 0)),
        ],
        out_specs=pl.BlockSpec((B, OUT_DIM), lambda c: (0, 0)),
        out_shape=jax.ShapeDtypeStruct((B, OUT_DIM), jnp.float32),
    )(g, dense, W_proj, b_proj2, W_sp, b_sp2, gammaB, betaB, W_headT, b_head2)


def kernel(indices, offsets, dense, emb_table, W_proj, b_proj, W_sp, b_sp,
           gamma, beta, W_head, b_head):
    del offsets
    g = _sc_gather(emb_table, indices.astype(jnp.int32))
    return _fused_dense(g, dense, W_proj, b_proj[None, :], W_sp,
                        b_sp[None, :], gamma[None, :], beta[None, :],
                        W_head.T, b_head[None, :])


# speedup vs baseline: 1.0860x; 1.0860x over previous
import functools

import jax
import jax.numpy as jnp
from jax import lax
from jax.experimental import pallas as pl
from jax.experimental.pallas import tpu as pltpu
from jax.experimental.pallas import tpu_sc as plsc

NUM_EMB = 100000
EMB_DIM = 128
PROJ_DIM = 256
DENSE_DIM = 64
CH = 32
HW = 256
OUT_DIM = 64
B = 4096
EPS = 1e-05

_NUM_SC = 2
_NUM_SUBCORES = 16
_NW = _NUM_SC * _NUM_SUBCORES


def _sc_gather(table, idx):
    b_per_w = B // _NW
    mesh = plsc.VectorSubcoreMesh(core_axis_name="c", subcore_axis_name="s")

    @functools.partial(
        pl.kernel,
        mesh=mesh,
        out_type=jax.ShapeDtypeStruct((B, EMB_DIM), jnp.float32),
        scratch_types=[
            pltpu.VMEM((b_per_w,), jnp.int32),
            pltpu.VMEM((b_per_w, EMB_DIM), jnp.float32),
            pltpu.SemaphoreType.DMA,
        ],
    )
    def gather_kernel(table_hbm, idx_hbm, out_hbm, idx_v, rows_v, sem):
        wid = lax.axis_index("s") * _NUM_SC + lax.axis_index("c")
        base = wid * b_per_w
        pltpu.sync_copy(idx_hbm.at[pl.ds(base, b_per_w)], idx_v)
        pltpu.async_copy(table_hbm.at[idx_v], rows_v, sem).wait()
        pltpu.sync_copy(rows_v, out_hbm.at[pl.ds(base, b_per_w)])

    return gather_kernel(table, idx)


_CR = 128
_NCH = B // _CR


def _fused_body(gd_ref, wproj_ref, bprojT_ref, wsp_ref, bsp_ref,
                gam_ref, bet_ref, whT_ref, bh_ref, out_ref, s_ref, acc_ref):
    c = pl.program_id(0)

    @pl.when(c == 0)
    def _():
        acc_ref[...] = jnp.zeros((B, OUT_DIM), jnp.float32)
    w_blk = wsp_ref[...]
    wp = w_blk[:, :PROJ_DIM]
    wd = w_blk[:, PROJ_DIM:]
    m = lax.dot_general(wp.astype(jnp.bfloat16),
                        wproj_ref[...].astype(jnp.bfloat16),
                        (((1,), (0,)), ((), ())),
                        preferred_element_type=jnp.float32
                        ).astype(jnp.bfloat16)
    rcol = lax.dot_general(wp, bprojT_ref[...], (((1,), (0,)), ((), ())),
                           preferred_element_type=jnp.float32) + bsp_ref[...]
    mfull = jnp.concatenate(
        [m, wd.astype(jnp.bfloat16), rcol.astype(jnp.bfloat16),
         jnp.zeros((HW, 63), jnp.bfloat16)], axis=1)
    gam_c = gam_ref[0, c]
    bg = bet_ref[0, c] / gam_c
    whrow = whT_ref[pl.ds(c, 1), :]
    inv_hw = 1.0 / HW
    s_ref[...] = lax.dot_general(gd_ref[...], mfull,
                                 (((1,), (1,)), ((), ())),
                                 preferred_element_type=jnp.float32)
    for i in range(_NCH):
        r0 = i * _CR
        sc = s_ref[pl.ds(r0, _CR), :]
        mu = jnp.sum(sc, axis=1) * inv_hw
        ms2 = jnp.sum(sc * sc, axis=1) * inv_hw
        var = ms2 - mu * mu
        a0 = lax.rsqrt(var + EPS)
        sd = (var + EPS) * a0
        theta = mu - bg * sd
        theta2 = theta[:, None]
        mm = jnp.sum(jnp.maximum(sc, theta2), axis=1) * inv_hw
        pooled = (gam_c * a0) * (mm - theta)
        acc_ref[pl.ds(r0, _CR), :] += pooled[:, None] * whrow

    @pl.when(c == CH - 1)
    def _():
        out_ref[...] = acc_ref[...] + bh_ref[...]


def _fused_dense(gd, W_proj, bprojT, W_sp, bsp2, gamma2, beta2,
                 W_headT, b_head2):
    return pl.pallas_call(
        _fused_body,
        grid=(CH,),
        in_specs=[
            pl.BlockSpec((B, PROJ_DIM), lambda c: (0, 0)),
            pl.BlockSpec((PROJ_DIM, EMB_DIM), lambda c: (0, 0)),
            pl.BlockSpec((PROJ_DIM, 1), lambda c: (0, 0)),
            pl.BlockSpec((HW, PROJ_DIM + DENSE_DIM), lambda c: (c, 0)),
            pl.BlockSpec((HW, 1), lambda c: (c, 0)),
            pl.BlockSpec(memory_space=pltpu.SMEM),
            pl.BlockSpec(memory_space=pltpu.SMEM),
            pl.BlockSpec((CH, OUT_DIM), lambda c: (0, 0)),
            pl.BlockSpec((1, OUT_DIM), lambda c: (0, 0)),
        ],
        out_specs=pl.BlockSpec((B, OUT_DIM), lambda c: (0, 0)),
        out_shape=jax.ShapeDtypeStruct((B, OUT_DIM), jnp.float32),
        scratch_shapes=[pltpu.VMEM((B, HW), jnp.float32),
                        pltpu.VMEM((B, OUT_DIM), jnp.float32)],
    )(gd, W_proj, bprojT, W_sp, bsp2, gamma2, beta2, W_headT, b_head2)


def kernel(indices, offsets, dense, emb_table, W_proj, b_proj, W_sp, b_sp,
           gamma, beta, W_head, b_head):
    del offsets
    g = _sc_gather(emb_table, indices.astype(jnp.int32))
    gd = jnp.concatenate(
        [g.astype(jnp.bfloat16), dense.astype(jnp.bfloat16),
         jnp.ones((B, 1), jnp.bfloat16), jnp.zeros((B, 63), jnp.bfloat16)],
        axis=1)
    return _fused_dense(gd, W_proj, b_proj[:, None], W_sp,
                        b_sp[:, None], gamma[None, :], beta[None, :],
                        W_head.T, b_head[None, :])
